# Initial kernel scaffold; baseline (speedup 1.0000x reference)
import jax
import jax.numpy as jnp
from jax.experimental import pallas as pl
from jax.experimental.pallas import tpu as pltpu

_N = 3136
_BLK = 784
_D = 256
_K = 1024
_PATCH = 768

_HI = jax.lax.Precision.HIGHEST


def _vq_block(p_ref, we_ref, eb_ref, w_ref, wt_ref, wd_ref, db_ref,
              z_ref, q_ref, r_ref, idx_ref):
    p = p_ref[...]
    z = jnp.dot(p, we_ref[...], preferred_element_type=jnp.float32,
                precision=_HI) + eb_ref[...]
    z_ref[...] = z
    w = w_ref[...]
    wsq = jnp.sum(w * w, axis=0, keepdims=True)
    d = wsq - 2.0 * jnp.dot(z, w, preferred_element_type=jnp.float32,
                            precision=_HI)
    am = jnp.argmin(d, axis=1).astype(jnp.int32)
    idx_ref[...] = am[:, None]
    onehot = (jax.lax.broadcasted_iota(jnp.int32, (_BLK, _K), 1)
              == am[:, None]).astype(jnp.float32)
    q = jnp.dot(onehot, wt_ref[...], preferred_element_type=jnp.float32,
                precision=_HI)
    q_ref[...] = q
    r_ref[...] = jnp.dot(q, wd_ref[...], preferred_element_type=jnp.float32,
                         precision=_HI) + db_ref[...]


def kernel(x, enc_W, enc_b, emb_weight, dec_W, dec_b):
    P = (x.reshape(16, 3, 14, 16, 14, 16)
          .transpose(0, 2, 4, 1, 3, 5)
          .reshape(_N, _PATCH))
    We = enc_W.reshape(_D, _PATCH).T
    eb = enc_b[None, :]
    wt = emb_weight.T
    Wd = (dec_W[:, :, ::-1, ::-1]
          .transpose(1, 0, 2, 3)
          .reshape(_D, _PATCH))
    db = jnp.repeat(dec_b, 256)[None, :]

    nblk = _N // _BLK
    z, q, r, idx = pl.pallas_call(
        _vq_block,
        grid=(nblk,),
        in_specs=[
            pl.BlockSpec((_BLK, _PATCH), lambda i: (i, 0)),
            pl.BlockSpec((_PATCH, _D), lambda i: (0, 0)),
            pl.BlockSpec((1, _D), lambda i: (0, 0)),
            pl.BlockSpec((_D, _K), lambda i: (0, 0)),
            pl.BlockSpec((_K, _D), lambda i: (0, 0)),
            pl.BlockSpec((_D, _PATCH), lambda i: (0, 0)),
            pl.BlockSpec((1, _PATCH), lambda i: (0, 0)),
        ],
        out_specs=[
            pl.BlockSpec((_BLK, _D), lambda i: (i, 0)),
            pl.BlockSpec((_BLK, _D), lambda i: (i, 0)),
            pl.BlockSpec((_BLK, _PATCH), lambda i: (i, 0)),
            pl.BlockSpec((_BLK, 1), lambda i: (i, 0)),
        ],
        out_shape=[
            jax.ShapeDtypeStruct((_N, _D), jnp.float32),
            jax.ShapeDtypeStruct((_N, _D), jnp.float32),
            jax.ShapeDtypeStruct((_N, _PATCH), jnp.float32),
            jax.ShapeDtypeStruct((_N, 1), jnp.int32),
        ],
    )(P, We, eb, emb_weight, wt, Wd, db)

    z_e = z.reshape(16, 14, 14, _D).transpose(0, 3, 1, 2)
    emb = q.reshape(16, 14, 14, _D).transpose(0, 3, 1, 2)
    recon = (r.reshape(16, 14, 14, 3, 16, 16)
              .transpose(0, 3, 1, 4, 2, 5)
              .reshape(16, 3, 224, 224))
    argmin = idx[:, 0].reshape(16, 14, 14)
    return (recon, z_e, emb, argmin)


# speedup vs baseline: 15.4814x; 15.4814x over previous
import jax
import jax.numpy as jnp
from jax.experimental import pallas as pl
from jax.experimental.pallas import tpu as pltpu

_N = 3136
_BLK = 784
_D = 256
_K = 1024
_PATCH = 768

_HI = jax.lax.Precision.HIGHEST


def _vq_block(p_ref, we_ref, eb_ref, w_ref, wt_ref, wd_ref, db_ref,
              z_ref, q_ref, r_ref, idx_ref):
    bf = jnp.bfloat16
    z = jnp.dot(p_ref[...].astype(bf), we_ref[...].astype(bf),
                preferred_element_type=jnp.float32) + eb_ref[...]
    z_ref[...] = z
    w = w_ref[...]
    wsq = jnp.sum(w * w, axis=0, keepdims=True)
    d = wsq - 2.0 * jnp.dot(z.astype(bf), w.astype(bf),
                            preferred_element_type=jnp.float32)
    am = jnp.argmin(d, axis=1).astype(jnp.int32)
    idx_ref[...] = am[:, None]
    onehot = (jax.lax.broadcasted_iota(jnp.int32, (_BLK, _K), 1)
              == am[:, None]).astype(jnp.float32)
    q = jnp.dot(onehot, wt_ref[...], preferred_element_type=jnp.float32,
                precision=_HI)
    q_ref[...] = q
    r_ref[...] = jnp.dot(q.astype(bf), wd_ref[...].astype(bf),
                         preferred_element_type=jnp.float32) + db_ref[...]


def kernel(x, enc_W, enc_b, emb_weight, dec_W, dec_b):
    P = (x.reshape(16, 3, 14, 16, 14, 16)
          .transpose(0, 2, 4, 1, 3, 5)
          .reshape(_N, _PATCH))
    We = enc_W.reshape(_D, _PATCH).T
    eb = enc_b[None, :]
    wt = emb_weight.T
    Wd = (dec_W[:, :, ::-1, ::-1]
          .transpose(1, 0, 2, 3)
          .reshape(_D, _PATCH))
    db = jnp.repeat(dec_b, 256)[None, :]

    nblk = _N // _BLK
    z, q, r, idx = pl.pallas_call(
        _vq_block,
        grid=(nblk,),
        in_specs=[
            pl.BlockSpec((_BLK, _PATCH), lambda i: (i, 0)),
            pl.BlockSpec((_PATCH, _D), lambda i: (0, 0)),
            pl.BlockSpec((1, _D), lambda i: (0, 0)),
            pl.BlockSpec((_D, _K), lambda i: (0, 0)),
            pl.BlockSpec((_K, _D), lambda i: (0, 0)),
            pl.BlockSpec((_D, _PATCH), lambda i: (0, 0)),
            pl.BlockSpec((1, _PATCH), lambda i: (0, 0)),
        ],
        out_specs=[
            pl.BlockSpec((_BLK, _D), lambda i: (i, 0)),
            pl.BlockSpec((_BLK, _D), lambda i: (i, 0)),
            pl.BlockSpec((_BLK, _PATCH), lambda i: (i, 0)),
            pl.BlockSpec((_BLK, 1), lambda i: (i, 0)),
        ],
        out_shape=[
            jax.ShapeDtypeStruct((_N, _D), jnp.float32),
            jax.ShapeDtypeStruct((_N, _D), jnp.float32),
            jax.ShapeDtypeStruct((_N, _PATCH), jnp.float32),
            jax.ShapeDtypeStruct((_N, 1), jnp.int32),
        ],
    )(P, We, eb, emb_weight, wt, Wd, db)

    z_e = z.reshape(16, 14, 14, _D).transpose(0, 3, 1, 2)
    emb = q.reshape(16, 14, 14, _D).transpose(0, 3, 1, 2)
    recon = (r.reshape(16, 14, 14, 3, 16, 16)
              .transpose(0, 3, 1, 4, 2, 5)
              .reshape(16, 3, 224, 224))
    argmin = idx[:, 0].reshape(16, 14, 14)
    return (recon, z_e, emb, argmin)


# speedup vs baseline: 29.1995x; 1.8861x over previous
import jax
import jax.numpy as jnp
from jax.experimental import pallas as pl
from jax.experimental.pallas import tpu as pltpu

_B = 16
_BB = 2
_NB = _BB * 196
_D = 256
_K = 1024
_PATCH = 768

_HI = jax.lax.Precision.HIGHEST


def _vq_block(x_ref, we_ref, eb_ref, w_ref, dw_ref, db_ref,
              rec_ref, z_ref, q_ref, idx_ref):
    bf = jnp.bfloat16
    f32 = jnp.float32

    x6 = x_ref[...].reshape(_BB, 3, 14, 16, 14, 16)
    p = x6.transpose(0, 2, 4, 1, 3, 5).reshape(_NB, _PATCH)

    z = jax.lax.dot_general(p.astype(bf), we_ref[...].astype(bf),
                            (((1,), (1,)), ((), ())),
                            preferred_element_type=f32) + eb_ref[...]
    z_ref[...] = z.reshape(_BB, 196, _D).transpose(0, 2, 1).reshape(
        _BB, _D, 14, 14)

    w = w_ref[...]
    wsq = jnp.sum(w * w, axis=0, keepdims=True)
    d = wsq - 2.0 * jnp.dot(z.astype(bf), w.astype(bf),
                            preferred_element_type=f32)
    am = jnp.argmin(d, axis=1).astype(jnp.int32)
    idx_ref[...] = am.reshape(_BB, 14, 14)

    onehot = (jax.lax.broadcasted_iota(jnp.int32, (_NB, _K), 1)
              == am[:, None]).astype(f32)
    q = jax.lax.dot_general(onehot, w, (((1,), (1,)), ((), ())),
                            preferred_element_type=f32, precision=_HI)
    q_ref[...] = q.reshape(_BB, 196, _D).transpose(0, 2, 1).reshape(
        _BB, _D, 14, 14)

    ri = jax.lax.broadcasted_iota(jnp.int32, (_D, _D), 0)
    ci = jax.lax.broadcasted_iota(jnp.int32, (_D, _D), 1)
    rev = (ri + ci == _D - 1).astype(f32)
    q16 = q.astype(bf)
    for c in range(3):
        rc = jnp.dot(q16, dw_ref[c].astype(bf), preferred_element_type=f32)
        rc = jnp.dot(rc, rev, preferred_element_type=f32, precision=_HI)
        rc = rc + db_ref[c:c + 1, :]
        rec_ref[:, c, :, :] = (rc.reshape(_BB, 14, 14, 16, 16)
                                 .transpose(0, 1, 3, 2, 4)
                                 .reshape(_BB, 224, 224))


def kernel(x, enc_W, enc_b, emb_weight, dec_W, dec_b):
    xr = x.reshape(_B, 3, 14, 16, 224)
    we = enc_W.reshape(_D, _PATCH)
    eb = enc_b[None, :]
    dw = dec_W.reshape(3, _D, _D)
    db = jnp.broadcast_to(dec_b[:, None], (3, _D))

    nblk = _B // _BB
    rec, z, q, idx = pl.pallas_call(
        _vq_block,
        grid=(nblk,),
        in_specs=[
            pl.BlockSpec((_BB, 3, 14, 16, 224), lambda i: (i, 0, 0, 0, 0)),
            pl.BlockSpec((_D, _PATCH), lambda i: (0, 0)),
            pl.BlockSpec((1, _D), lambda i: (0, 0)),
            pl.BlockSpec((_D, _K), lambda i: (0, 0)),
            pl.BlockSpec((3, _D, _D), lambda i: (0, 0, 0)),
            pl.BlockSpec((3, _D), lambda i: (0, 0)),
        ],
        out_specs=[
            pl.BlockSpec((_BB, 3, 224, 224), lambda i: (i, 0, 0, 0)),
            pl.BlockSpec((_BB, _D, 14, 14), lambda i: (i, 0, 0, 0)),
            pl.BlockSpec((_BB, _D, 14, 14), lambda i: (i, 0, 0, 0)),
            pl.BlockSpec((_BB, 14, 14), lambda i: (i, 0, 0)),
        ],
        out_shape=[
            jax.ShapeDtypeStruct((_B, 3, 224, 224), jnp.float32),
            jax.ShapeDtypeStruct((_B, _D, 14, 14), jnp.float32),
            jax.ShapeDtypeStruct((_B, _D, 14, 14), jnp.float32),
            jax.ShapeDtypeStruct((_B, 14, 14), jnp.int32),
        ],
    )(xr, we, eb, emb_weight, dw, db)
    return (rec, z, q, idx)


# speedup vs baseline: 30.3627x; 1.0398x over previous
import functools
import jax
import jax.numpy as jnp
from jax import lax
from jax.experimental import pallas as pl
from jax.experimental.pallas import tpu as pltpu
from jax.experimental.pallas import tpu_sc as plsc

_B = 16
_BB = 2
_NB = _BB * 196
_D = 256
_K = 1024
_PATCH = 768
_N = 3136
_NPAD = 3328
_NC, _NS = 2, 16
_BPW = _NPAD // (_NC * _NS)

_HI = jax.lax.Precision.HIGHEST


def _enc_block(x_ref, we_ref, eb_ref, w_ref,
               z_ref, idx_ref, wt_ref):
    bf = jnp.bfloat16
    f32 = jnp.float32
    i = pl.program_id(0)

    x6 = x_ref[...].astype(bf).reshape(_BB, 3, 14, 16, 14, 16)
    p = x6.transpose(0, 2, 4, 1, 3, 5).reshape(_NB, _PATCH)
    z = jax.lax.dot_general(p, we_ref[...].astype(bf),
                            (((1,), (1,)), ((), ())),
                            preferred_element_type=f32) + eb_ref[...]
    z_ref[...] = z.reshape(_BB, 196, _D).transpose(0, 2, 1).reshape(
        _BB, _D, 14, 14)

    w = w_ref[...]
    wsq = jnp.sum(w * w, axis=0, keepdims=True)
    d = wsq - 2.0 * jnp.dot(z.astype(bf), w.astype(bf),
                            preferred_element_type=f32)
    am = jnp.argmin(d, axis=1).astype(jnp.int32)
    idx_ref[...] = am.reshape(_BB, 14, 14)

    @pl.when(i == 0)
    def _():
        wt_ref[...] = w.T


def _dec_block(q_ref, dw_ref, db_ref, rec_ref, emb_ref):
    bf = jnp.bfloat16
    f32 = jnp.float32
    q = q_ref[...]
    emb_ref[...] = q.reshape(_BB, 196, _D).transpose(0, 2, 1).reshape(
        _BB, _D, 14, 14)
    ri = jax.lax.broadcasted_iota(jnp.int32, (_D, _D), 0)
    ci = jax.lax.broadcasted_iota(jnp.int32, (_D, _D), 1)
    rev = (ri + ci == _D - 1).astype(f32)
    q16 = q.astype(bf)
    for c in range(3):
        rc = jnp.dot(q16, dw_ref[c].astype(bf), preferred_element_type=f32)
        rc = jnp.dot(rc, rev, preferred_element_type=f32, precision=_HI)
        rc = rc + db_ref[c:c + 1, :]
        rec_ref[:, c, :, :] = (rc.reshape(_BB, 14, 14, 16, 16)
                                 .transpose(0, 1, 3, 2, 4)
                                 .reshape(_BB, 224, 224))


def _sc_gather(table_hbm, idx_hbm, out_hbm, idx_v, rows_v, sem):
    wid = lax.axis_index("s") * _NC + lax.axis_index("c")
    base = wid * _BPW
    pltpu.sync_copy(idx_hbm.at[pl.ds(base, _BPW)], idx_v)
    pltpu.async_copy(table_hbm.at[idx_v], rows_v, sem).wait()
    pltpu.sync_copy(rows_v, out_hbm.at[pl.ds(base, _BPW)])


def kernel(x, enc_W, enc_b, emb_weight, dec_W, dec_b):
    xr = x.reshape(_B, 3, 14, 16, 224)
    we = enc_W.reshape(_D, _PATCH)
    eb = enc_b[None, :]
    dw = dec_W.reshape(3, _D, _D)
    db = jnp.broadcast_to(dec_b[:, None], (3, _D))

    nblk = _B // _BB
    z, idx, wt = pl.pallas_call(
        _enc_block,
        grid=(nblk,),
        in_specs=[
            pl.BlockSpec((_BB, 3, 14, 16, 224), lambda i: (i, 0, 0, 0, 0)),
            pl.BlockSpec((_D, _PATCH), lambda i: (0, 0)),
            pl.BlockSpec((1, _D), lambda i: (0, 0)),
            pl.BlockSpec((_D, _K), lambda i: (0, 0)),
        ],
        out_specs=[
            pl.BlockSpec((_BB, _D, 14, 14), lambda i: (i, 0, 0, 0)),
            pl.BlockSpec((_BB, 14, 14), lambda i: (i, 0, 0)),
            pl.BlockSpec((_K, _D), lambda i: (0, 0)),
        ],
        out_shape=[
            jax.ShapeDtypeStruct((_B, _D, 14, 14), jnp.float32),
            jax.ShapeDtypeStruct((_B, 14, 14), jnp.int32),
            jax.ShapeDtypeStruct((_K, _D), jnp.float32),
        ],
    )(xr, we, eb, emb_weight)

    idx_flat = jnp.concatenate(
        [idx.reshape(_N), jnp.zeros((_NPAD - _N,), jnp.int32)])

    mesh = plsc.VectorSubcoreMesh(core_axis_name="c", subcore_axis_name="s")
    qflat = pl.kernel(
        _sc_gather,
        out_type=jax.ShapeDtypeStruct((_NPAD, _D), jnp.float32),
        mesh=mesh,
        scratch_types=[
            pltpu.VMEM((_BPW,), jnp.int32),
            pltpu.VMEM((_BPW, _D), jnp.float32),
            pltpu.SemaphoreType.DMA,
        ],
    )(wt, idx_flat)

    rec, emb = pl.pallas_call(
        _dec_block,
        grid=(nblk,),
        in_specs=[
            pl.BlockSpec((_NB, _D), lambda i: (i, 0)),
            pl.BlockSpec((3, _D, _D), lambda i: (0, 0, 0)),
            pl.BlockSpec((3, _D), lambda i: (0, 0)),
        ],
        out_specs=[
            pl.BlockSpec((_BB, 3, 224, 224), lambda i: (i, 0, 0, 0)),
            pl.BlockSpec((_BB, _D, 14, 14), lambda i: (i, 0, 0, 0)),
        ],
        out_shape=[
            jax.ShapeDtypeStruct((_B, 3, 224, 224), jnp.float32),
            jax.ShapeDtypeStruct((_B, _D, 14, 14), jnp.float32),
        ],
    )(qflat, dw, db)
    return (rec, z, emb, idx)
